# baseline (speedup 1.0000x reference)
import functools

import jax
import jax.numpy as jnp
from jax import lax
from jax.experimental import pallas as pl
from jax.experimental.pallas import tpu as pltpu
from jax.experimental.pallas import tpu_sc as plsc

N = 100000
D = 128
NUM_SEG = 256
EPS = 1e-6
LANES = 16

GROUP = 64
FULL_GROUPS = N // GROUP
REM = N - FULL_GROUPS * GROUP
NW = 32
BASE_G = FULL_GROUPS // NW
EXTRA = FULL_GROUPS - BASE_G * NW
MAXG = BASE_G + 1

R = 1000
GRID = N // R
WIN = 32
WPAD = WIN


def _sc_stats(x, batch, batch_tail):
    mesh = plsc.VectorSubcoreMesh(core_axis_name="c", subcore_axis_name="s")

    @functools.partial(
        pl.kernel,
        mesh=mesh,
        out_type=[
            jax.ShapeDtypeStruct((NW, NUM_SEG, D), jnp.float32),
            jax.ShapeDtypeStruct((NW, NUM_SEG, D), jnp.float32),
            jax.ShapeDtypeStruct((NW, NUM_SEG, LANES), jnp.float32),
        ],
        scratch_types=[
            pltpu.VMEM((GROUP, D), jnp.float32),
            pltpu.VMEM((GROUP, D), jnp.float32),
            pltpu.VMEM((2, GROUP), jnp.int32),
            pltpu.VMEM((REM,), jnp.int32),
            pltpu.VMEM((NUM_SEG, D), jnp.float32),
            pltpu.VMEM((NUM_SEG, D), jnp.float32),
            pltpu.VMEM((NUM_SEG, LANES), jnp.float32),
            pltpu.SemaphoreType.DMA,
            pltpu.SemaphoreType.DMA,
        ],
    )
    def k(x_hbm, b_hbm, btail_hbm, sums_o, sqs_o, cnts_o,
          xv0, xv1, idx2, idx_rem, sum_v, sq_v, cnt_v, lsem0, lsem1):
        cid = lax.axis_index("c")
        sid = lax.axis_index("s")
        wid = cid * 16 + sid

        zero = jnp.zeros((LANES,), jnp.float32)

        def zrow(r, carry):
            cnt_v[r, :] = zero
            for j in range(D // LANES):
                sum_v[r, pl.ds(j * LANES, LANES)] = zero
                sq_v[r, pl.ds(j * LANES, LANES)] = zero
            return carry

        lax.fori_loop(0, NUM_SEG, zrow, 0)

        n_g = jnp.where(wid < EXTRA, MAXG, BASE_G)
        g0 = wid * BASE_G + jnp.minimum(wid, EXTRA)

        def start_load(xv, lsem, slot, t):
            off = (g0 + t) * GROUP
            pltpu.async_copy(x_hbm.at[pl.ds(off, GROUP)], xv, lsem)
            pltpu.async_copy(b_hbm.at[pl.ds(off, GROUP)], idx2.at[slot], lsem)

        def wait_load(xv, lsem, slot):
            pltpu.make_async_copy(x_hbm.at[pl.ds(0, GROUP)], xv, lsem).wait()
            pltpu.make_async_copy(b_hbm.at[pl.ds(0, GROUP)], idx2.at[slot],
                                  lsem).wait()

        def accum_row(xv, seg, r):
            plsc.addupdate(cnt_v.at[seg], jnp.ones((LANES,), jnp.float32))
            for j in range(D // LANES):
                v = xv[r, pl.ds(j * LANES, LANES)]
                plsc.addupdate(sum_v.at[seg, pl.ds(j * LANES, LANES)], v)
                plsc.addupdate(sq_v.at[seg, pl.ds(j * LANES, LANES)], v * v)

        def process(xv, slot):
            ids_head = idx2[slot, pl.ds(0, LANES)]
            ids_tail = idx2[slot, pl.ds(GROUP - LANES, LANES)]
            i0 = ids_head[0]
            i_last = ids_tail[LANES - 1]

            @pl.when(i0 == i_last)
            def _():
                def row(r, accs):
                    out = []
                    for j in range(D // LANES):
                        v = xv[r, pl.ds(j * LANES, LANES)]
                        out.append(accs[2 * j] + v)
                        out.append(accs[2 * j + 1] + v * v)
                    return tuple(out)

                init = tuple(jnp.zeros((LANES,), jnp.float32)
                             for _ in range(2 * (D // LANES)))
                accs = lax.fori_loop(0, GROUP, row, init)
                for j in range(D // LANES):
                    plsc.addupdate(sum_v.at[i0, pl.ds(j * LANES, LANES)],
                                   accs[2 * j])
                    plsc.addupdate(sq_v.at[i0, pl.ds(j * LANES, LANES)],
                                   accs[2 * j + 1])
                plsc.addupdate(cnt_v.at[i0],
                               jnp.full((LANES,), float(GROUP), jnp.float32))

            @pl.when(i0 != i_last)
            def _():
                def chunk(c, carry):
                    ids_v = idx2[slot, pl.ds(c * LANES, LANES)]
                    for l in range(LANES):
                        accum_row(xv, ids_v[l], c * LANES + l)
                    return carry

                lax.fori_loop(0, GROUP // LANES, chunk, 0)

        def step(t, cur, nxt):
            xv, lsem, slot = cur
            xv_n, lsem_n, slot_n = nxt

            @pl.when(t + 1 < n_g)
            def _():
                start_load(xv_n, lsem_n, slot_n, t + 1)

            @pl.when(t < n_g)
            def _():
                wait_load(xv, lsem, slot)
                process(xv, slot)

        start_load(xv0, lsem0, 0, 0)

        def pair(p, carry):
            step(2 * p, (xv0, lsem0, 0), (xv1, lsem1, 1))
            step(2 * p + 1, (xv1, lsem1, 1), (xv0, lsem0, 0))
            return carry

        lax.fori_loop(0, (MAXG + 1) // 2, pair, 0)

        @pl.when(wid == NW - 1)
        def _():
            pltpu.sync_copy(x_hbm.at[pl.ds(FULL_GROUPS * GROUP, REM)],
                            xv0.at[pl.ds(0, REM)])
            pltpu.sync_copy(btail_hbm, idx_rem)

            def chunk(c, carry):
                ids_v = idx_rem[pl.ds(c * LANES, LANES)]
                for l in range(LANES):
                    accum_row(xv0, ids_v[l], c * LANES + l)
                return carry

            lax.fori_loop(0, REM // LANES, chunk, 0)

        pltpu.sync_copy(sum_v, sums_o.at[wid])
        pltpu.sync_copy(sq_v, sqs_o.at[wid])
        pltpu.sync_copy(cnt_v, cnts_o.at[wid])

    return k(x, batch, batch_tail)


def _tc_norm(x, batch3, sums, sqs, cnts, w2, b2, ms2):
    def body(x_ref, b_ref, sums_ref, sqs_ref, cnts_ref, w_ref, bi_ref, ms_ref,
             o_ref, hi_scr, lo_scr):
        i = pl.program_id(0)

        @pl.when(i == 0)
        def _():
            sums_c = jnp.sum(sums_ref[...], axis=0)
            sqs_c = jnp.sum(sqs_ref[...], axis=0)
            cnt = jnp.sum(cnts_ref[...], axis=0)[:, 0:1]
            nc = jnp.maximum(cnt, 1.0)
            m = sums_c / nc
            s = ms_ref[...]
            seg_sq = sqs_c - nc * m * m * s * (2.0 - s)
            var = jnp.maximum(seg_sq, 0.0) / nc
            a = w_ref[...] * lax.rsqrt(var + EPS)
            bt = bi_ref[...] - a * s * m
            ab = jnp.concatenate([a, bt], axis=1)
            hi = ab.astype(jnp.bfloat16)
            lo = (ab - hi.astype(jnp.float32)).astype(jnp.bfloat16)
            hi_scr[pl.ds(0, NUM_SEG), :] = hi
            lo_scr[pl.ds(0, NUM_SEG), :] = lo
            pad = jnp.zeros((WPAD, 2 * D), jnp.bfloat16)
            hi_scr[pl.ds(NUM_SEG, WPAD), :] = pad
            lo_scr[pl.ds(NUM_SEG, WPAD), :] = pad

        b = b_ref[0]
        b0 = b_ref[0, 0, 0]
        blast = b_ref[0, R - 1, 0]
        base8 = pl.multiple_of((b0 >> 4) << 4, 16)

        @pl.when(blast - base8 < WIN)
        def _():
            oh = (lax.broadcasted_iota(jnp.int32, (R, WIN), 1) + base8 == b)
            oh = oh.astype(jnp.bfloat16)
            hiw = hi_scr[pl.ds(base8, WIN), :]
            low = lo_scr[pl.ds(base8, WIN), :]
            g = jax.lax.dot(oh, hiw, preferred_element_type=jnp.float32)
            g += jax.lax.dot(oh, low, preferred_element_type=jnp.float32)
            o_ref[...] = x_ref[...] * g[:, :D] + g[:, D:]

        @pl.when(blast - base8 >= WIN)
        def _():
            oh = (lax.broadcasted_iota(jnp.int32, (R, NUM_SEG), 1) == b)
            oh = oh.astype(jnp.bfloat16)
            hif = hi_scr[pl.ds(0, NUM_SEG), :]
            lof = lo_scr[pl.ds(0, NUM_SEG), :]
            g = jax.lax.dot(oh, hif, preferred_element_type=jnp.float32)
            g += jax.lax.dot(oh, lof, preferred_element_type=jnp.float32)
            o_ref[...] = x_ref[...] * g[:, :D] + g[:, D:]

    return pl.pallas_call(
        body,
        grid=(GRID,),
        in_specs=[
            pl.BlockSpec((R, D), lambda i: (i, 0)),
            pl.BlockSpec((1, R, 1), lambda i: (i, 0, 0)),
            pl.BlockSpec((NW, NUM_SEG, D), lambda i: (0, 0, 0)),
            pl.BlockSpec((NW, NUM_SEG, D), lambda i: (0, 0, 0)),
            pl.BlockSpec((NW, NUM_SEG, LANES), lambda i: (0, 0, 0)),
            pl.BlockSpec((1, D), lambda i: (0, 0)),
            pl.BlockSpec((1, D), lambda i: (0, 0)),
            pl.BlockSpec((1, D), lambda i: (0, 0)),
        ],
        out_specs=pl.BlockSpec((R, D), lambda i: (i, 0)),
        out_shape=jax.ShapeDtypeStruct((N, D), jnp.float32),
        scratch_shapes=[
            pltpu.VMEM((NUM_SEG + WPAD, 2 * D), jnp.bfloat16),
            pltpu.VMEM((NUM_SEG + WPAD, 2 * D), jnp.bfloat16),
        ],
        compiler_params=pltpu.CompilerParams(
            dimension_semantics=("arbitrary",)),
    )(x, batch3, sums, sqs, cnts, w2, b2, ms2)


def kernel(x, batch, weight, bias, mean_scale):
    batch_tail = batch[FULL_GROUPS * GROUP:]
    sums, sqs, cnts = _sc_stats(x, batch, batch_tail)
    batch3 = batch.reshape(GRID, R, 1)
    return _tc_norm(x, batch3, sums, sqs, cnts,
                    weight.reshape(1, D), bias.reshape(1, D),
                    mean_scale.reshape(1, D))


# speedup vs baseline: 1.1165x; 1.1165x over previous
import functools

import jax
import jax.numpy as jnp
from jax import lax
from jax.experimental import pallas as pl
from jax.experimental.pallas import tpu as pltpu
from jax.experimental.pallas import tpu_sc as plsc

N = 100000
D = 128
NUM_SEG = 256
EPS = 1e-6
LANES = 16

GROUP = 64
FULL_GROUPS = N // GROUP
REM = N - FULL_GROUPS * GROUP
NW = 32
BASE_G = FULL_GROUPS // NW
EXTRA = FULL_GROUPS - BASE_G * NW
MAXG = BASE_G + 1

R = 1000
GRID = N // R
WIN = 32
WPAD = WIN


def _sc_stats(x, batch, batch_tail):
    mesh = plsc.VectorSubcoreMesh(core_axis_name="c", subcore_axis_name="s")

    @functools.partial(
        pl.kernel,
        mesh=mesh,
        out_type=[
            jax.ShapeDtypeStruct((NW, NUM_SEG, D), jnp.float32),
            jax.ShapeDtypeStruct((NW, NUM_SEG, D), jnp.float32),
            jax.ShapeDtypeStruct((NW, NUM_SEG, LANES), jnp.float32),
        ],
        scratch_types=[
            pltpu.VMEM((GROUP, D), jnp.float32),
            pltpu.VMEM((GROUP, D), jnp.float32),
            pltpu.VMEM((2, GROUP), jnp.int32),
            pltpu.VMEM((REM,), jnp.int32),
            pltpu.VMEM((NUM_SEG, D), jnp.float32),
            pltpu.VMEM((NUM_SEG, D), jnp.float32),
            pltpu.VMEM((NUM_SEG, LANES), jnp.float32),
            pltpu.SemaphoreType.DMA,
            pltpu.SemaphoreType.DMA,
        ],
    )
    def k(x_hbm, b_hbm, btail_hbm, sums_o, sqs_o, cnts_o,
          xv0, xv1, idx2, idx_rem, sum_v, sq_v, cnt_v, lsem0, lsem1):
        cid = lax.axis_index("c")
        sid = lax.axis_index("s")
        wid = cid * 16 + sid

        zero = jnp.zeros((LANES,), jnp.float32)

        def zrow(r, carry):
            cnt_v[r, :] = zero
            for j in range(D // LANES):
                sum_v[r, pl.ds(j * LANES, LANES)] = zero
                sq_v[r, pl.ds(j * LANES, LANES)] = zero
            return carry

        lax.fori_loop(0, NUM_SEG, zrow, 0)

        n_g = jnp.where(wid < EXTRA, MAXG, BASE_G)
        g0 = wid * BASE_G + jnp.minimum(wid, EXTRA)

        def start_load(xv, lsem, slot, t):
            off = (g0 + t) * GROUP
            pltpu.async_copy(x_hbm.at[pl.ds(off, GROUP)], xv, lsem)
            pltpu.async_copy(b_hbm.at[pl.ds(off, GROUP)], idx2.at[slot], lsem)

        def wait_load(xv, lsem, slot):
            pltpu.make_async_copy(x_hbm.at[pl.ds(0, GROUP)], xv, lsem).wait()
            pltpu.make_async_copy(b_hbm.at[pl.ds(0, GROUP)], idx2.at[slot],
                                  lsem).wait()

        def accum_row(xv, seg, r):
            plsc.addupdate(cnt_v.at[seg], jnp.ones((LANES,), jnp.float32))
            for j in range(D // LANES):
                v = xv[r, pl.ds(j * LANES, LANES)]
                plsc.addupdate(sum_v.at[seg, pl.ds(j * LANES, LANES)], v)
                plsc.addupdate(sq_v.at[seg, pl.ds(j * LANES, LANES)], v * v)

        def process(xv, slot):
            ids_head = idx2[slot, pl.ds(0, LANES)]
            ids_tail = idx2[slot, pl.ds(GROUP - LANES, LANES)]
            i0 = ids_head[0]
            i_last = ids_tail[LANES - 1]

            @pl.when(i0 == i_last)
            def _():
                def row(r, accs):
                    out = []
                    for j in range(D // LANES):
                        v = xv[r, pl.ds(j * LANES, LANES)]
                        out.append(accs[2 * j] + v)
                        out.append(accs[2 * j + 1] + v * v)
                    return tuple(out)

                init = tuple(jnp.zeros((LANES,), jnp.float32)
                             for _ in range(2 * (D // LANES)))
                accs = lax.fori_loop(0, GROUP, row, init)
                for j in range(D // LANES):
                    plsc.addupdate(sum_v.at[i0, pl.ds(j * LANES, LANES)],
                                   accs[2 * j])
                    plsc.addupdate(sq_v.at[i0, pl.ds(j * LANES, LANES)],
                                   accs[2 * j + 1])
                plsc.addupdate(cnt_v.at[i0],
                               jnp.full((LANES,), float(GROUP), jnp.float32))

            @pl.when(i0 != i_last)
            def _():
                def chunk(c, carry):
                    ids_v = idx2[slot, pl.ds(c * LANES, LANES)]
                    for l in range(LANES):
                        accum_row(xv, ids_v[l], c * LANES + l)
                    return carry

                lax.fori_loop(0, GROUP // LANES, chunk, 0)

        def step(t, cur, nxt):
            xv, lsem, slot = cur
            xv_n, lsem_n, slot_n = nxt

            @pl.when(t + 1 < n_g)
            def _():
                start_load(xv_n, lsem_n, slot_n, t + 1)

            @pl.when(t < n_g)
            def _():
                wait_load(xv, lsem, slot)
                process(xv, slot)

        start_load(xv0, lsem0, 0, 0)

        def pair(p, carry):
            step(2 * p, (xv0, lsem0, 0), (xv1, lsem1, 1))
            step(2 * p + 1, (xv1, lsem1, 1), (xv0, lsem0, 0))
            return carry

        lax.fori_loop(0, (MAXG + 1) // 2, pair, 0)

        @pl.when(wid == NW - 1)
        def _():
            pltpu.sync_copy(x_hbm.at[pl.ds(FULL_GROUPS * GROUP, REM)],
                            xv0.at[pl.ds(0, REM)])
            pltpu.sync_copy(btail_hbm, idx_rem)

            def chunk(c, carry):
                ids_v = idx_rem[pl.ds(c * LANES, LANES)]
                for l in range(LANES):
                    accum_row(xv0, ids_v[l], c * LANES + l)
                return carry

            lax.fori_loop(0, REM // LANES, chunk, 0)

        pltpu.sync_copy(sum_v, sums_o.at[wid])
        pltpu.sync_copy(sq_v, sqs_o.at[wid])
        pltpu.sync_copy(cnt_v, cnts_o.at[wid])

    return k(x, batch, batch_tail)


def _tc_norm(x, batch3, sums, sqs, cnts, w2, b2, ms2):
    def body(x_ref, b_ref, sums_ref, sqs_ref, cnts_ref, w_ref, bi_ref, ms_ref,
             o_ref, hi_scr, lo_scr):
        i = pl.program_id(0)

        @pl.when(i == 0)
        def _():
            sums_c = jnp.sum(sums_ref[...], axis=0)
            sqs_c = jnp.sum(sqs_ref[...], axis=0)
            cnt = jnp.sum(cnts_ref[...], axis=0)[:, 0:1]
            nc = jnp.maximum(cnt, 1.0)
            m = sums_c / nc
            s = ms_ref[...]
            seg_sq = sqs_c - nc * m * m * s * (2.0 - s)
            var = jnp.maximum(seg_sq, 0.0) / nc
            a = w_ref[...] * lax.rsqrt(var + EPS)
            bt = bi_ref[...] - a * s * m
            ab = jnp.concatenate([a, bt], axis=1)
            hi = ab.astype(jnp.bfloat16)
            lo = (ab - hi.astype(jnp.float32)).astype(jnp.bfloat16)
            hi_scr[pl.ds(0, NUM_SEG), :] = hi
            lo_scr[pl.ds(0, NUM_SEG), :] = lo
            pad = jnp.zeros((WPAD, 2 * D), jnp.bfloat16)
            hi_scr[pl.ds(NUM_SEG, WPAD), :] = pad
            lo_scr[pl.ds(NUM_SEG, WPAD), :] = pad

        bm = b_ref[0]
        b0 = b_ref[0, 0, 0]
        blast = b_ref[0, 0, R - 1]
        base16 = pl.multiple_of((b0 >> 4) << 4, 16)
        tdims = (((0,), (0,)), ((), ()))

        @pl.when(blast - base16 < WIN)
        def _():
            oht = (lax.broadcasted_iota(jnp.int32, (WIN, R), 0) + base16 == bm)
            oht = oht.astype(jnp.bfloat16)
            hiw = hi_scr[pl.ds(base16, WIN), :]
            low = lo_scr[pl.ds(base16, WIN), :]
            g = lax.dot_general(oht, hiw, tdims,
                                preferred_element_type=jnp.float32)
            g += lax.dot_general(oht, low, tdims,
                                 preferred_element_type=jnp.float32)
            o_ref[...] = x_ref[...] * g[:, :D] + g[:, D:]

        @pl.when(blast - base16 >= WIN)
        def _():
            oht = (lax.broadcasted_iota(jnp.int32, (NUM_SEG, R), 0) == bm)
            oht = oht.astype(jnp.bfloat16)
            hif = hi_scr[pl.ds(0, NUM_SEG), :]
            lof = lo_scr[pl.ds(0, NUM_SEG), :]
            g = lax.dot_general(oht, hif, tdims,
                                preferred_element_type=jnp.float32)
            g += lax.dot_general(oht, lof, tdims,
                                 preferred_element_type=jnp.float32)
            o_ref[...] = x_ref[...] * g[:, :D] + g[:, D:]

    return pl.pallas_call(
        body,
        grid=(GRID,),
        in_specs=[
            pl.BlockSpec((R, D), lambda i: (i, 0)),
            pl.BlockSpec((1, 1, R), lambda i: (i, 0, 0)),
            pl.BlockSpec((NW, NUM_SEG, D), lambda i: (0, 0, 0)),
            pl.BlockSpec((NW, NUM_SEG, D), lambda i: (0, 0, 0)),
            pl.BlockSpec((NW, NUM_SEG, LANES), lambda i: (0, 0, 0)),
            pl.BlockSpec((1, D), lambda i: (0, 0)),
            pl.BlockSpec((1, D), lambda i: (0, 0)),
            pl.BlockSpec((1, D), lambda i: (0, 0)),
        ],
        out_specs=pl.BlockSpec((R, D), lambda i: (i, 0)),
        out_shape=jax.ShapeDtypeStruct((N, D), jnp.float32),
        scratch_shapes=[
            pltpu.VMEM((NUM_SEG + WPAD, 2 * D), jnp.bfloat16),
            pltpu.VMEM((NUM_SEG + WPAD, 2 * D), jnp.bfloat16),
        ],
        compiler_params=pltpu.CompilerParams(
            dimension_semantics=("arbitrary",)),
    )(x, batch3, sums, sqs, cnts, w2, b2, ms2)


def kernel(x, batch, weight, bias, mean_scale):
    batch_tail = batch[FULL_GROUPS * GROUP:]
    sums, sqs, cnts = _sc_stats(x, batch, batch_tail)
    batch3 = batch.reshape(GRID, 1, R)
    return _tc_norm(x, batch3, sums, sqs, cnts,
                    weight.reshape(1, D), bias.reshape(1, D),
                    mean_scale.reshape(1, D))


# speedup vs baseline: 1.3415x; 1.2015x over previous
import functools

import jax
import jax.numpy as jnp
from jax import lax
from jax.experimental import pallas as pl
from jax.experimental.pallas import tpu as pltpu
from jax.experimental.pallas import tpu_sc as plsc

N = 100000
D = 128
NUM_SEG = 256
EPS = 1e-6
LANES = 16

GROUP = 64
FULL_GROUPS = N // GROUP
REM = N - FULL_GROUPS * GROUP
NW = 32
BASE_G = FULL_GROUPS // NW
EXTRA = FULL_GROUPS - BASE_G * NW
MAXG = BASE_G + 1

R = 2000
GRID = N // R
WIN = 32
WPAD = WIN


def _sc_stats(x, batch, batch_tail):
    mesh = plsc.VectorSubcoreMesh(core_axis_name="c", subcore_axis_name="s")

    @functools.partial(
        pl.kernel,
        mesh=mesh,
        out_type=[
            jax.ShapeDtypeStruct((NW, NUM_SEG, D), jnp.float32),
            jax.ShapeDtypeStruct((NW, NUM_SEG, D), jnp.float32),
            jax.ShapeDtypeStruct((NW, NUM_SEG, LANES), jnp.float32),
        ],
        scratch_types=[
            pltpu.VMEM((GROUP, D), jnp.float32),
            pltpu.VMEM((GROUP, D), jnp.float32),
            pltpu.VMEM((2, GROUP), jnp.int32),
            pltpu.VMEM((REM,), jnp.int32),
            pltpu.VMEM((NUM_SEG, D), jnp.float32),
            pltpu.VMEM((NUM_SEG, D), jnp.float32),
            pltpu.VMEM((NUM_SEG, LANES), jnp.float32),
            pltpu.SemaphoreType.DMA,
            pltpu.SemaphoreType.DMA,
        ],
    )
    def k(x_hbm, b_hbm, btail_hbm, sums_o, sqs_o, cnts_o,
          xv0, xv1, idx2, idx_rem, sum_v, sq_v, cnt_v, lsem0, lsem1):
        cid = lax.axis_index("c")
        sid = lax.axis_index("s")
        wid = cid * 16 + sid

        zero = jnp.zeros((LANES,), jnp.float32)

        def zrow(r, carry):
            cnt_v[r, :] = zero
            for j in range(D // LANES):
                sum_v[r, pl.ds(j * LANES, LANES)] = zero
                sq_v[r, pl.ds(j * LANES, LANES)] = zero
            return carry

        lax.fori_loop(0, NUM_SEG, zrow, 0)

        n_g = jnp.where(wid < EXTRA, MAXG, BASE_G)
        g0 = wid * BASE_G + jnp.minimum(wid, EXTRA)

        def start_load(xv, lsem, slot, t):
            off = (g0 + t) * GROUP
            pltpu.async_copy(x_hbm.at[pl.ds(off, GROUP)], xv, lsem)
            pltpu.async_copy(b_hbm.at[pl.ds(off, GROUP)], idx2.at[slot], lsem)

        def wait_load(xv, lsem, slot):
            pltpu.make_async_copy(x_hbm.at[pl.ds(0, GROUP)], xv, lsem).wait()
            pltpu.make_async_copy(b_hbm.at[pl.ds(0, GROUP)], idx2.at[slot],
                                  lsem).wait()

        def accum_row(xv, seg, r):
            plsc.addupdate(cnt_v.at[seg], jnp.ones((LANES,), jnp.float32))
            for j in range(D // LANES):
                v = xv[r, pl.ds(j * LANES, LANES)]
                plsc.addupdate(sum_v.at[seg, pl.ds(j * LANES, LANES)], v)
                plsc.addupdate(sq_v.at[seg, pl.ds(j * LANES, LANES)], v * v)

        def process(xv, slot):
            ids_head = idx2[slot, pl.ds(0, LANES)]
            ids_tail = idx2[slot, pl.ds(GROUP - LANES, LANES)]
            i0 = ids_head[0]
            i_last = ids_tail[LANES - 1]

            @pl.when(i0 == i_last)
            def _():
                def row(r, accs):
                    out = []
                    for j in range(D // LANES):
                        v = xv[r, pl.ds(j * LANES, LANES)]
                        out.append(accs[2 * j] + v)
                        out.append(accs[2 * j + 1] + v * v)
                    return tuple(out)

                init = tuple(jnp.zeros((LANES,), jnp.float32)
                             for _ in range(2 * (D // LANES)))
                accs = lax.fori_loop(0, GROUP, row, init)
                for j in range(D // LANES):
                    plsc.addupdate(sum_v.at[i0, pl.ds(j * LANES, LANES)],
                                   accs[2 * j])
                    plsc.addupdate(sq_v.at[i0, pl.ds(j * LANES, LANES)],
                                   accs[2 * j + 1])
                plsc.addupdate(cnt_v.at[i0],
                               jnp.full((LANES,), float(GROUP), jnp.float32))

            @pl.when(i0 != i_last)
            def _():
                def chunk(c, carry):
                    ids_v = idx2[slot, pl.ds(c * LANES, LANES)]
                    for l in range(LANES):
                        accum_row(xv, ids_v[l], c * LANES + l)
                    return carry

                lax.fori_loop(0, GROUP // LANES, chunk, 0)

        def step(t, cur, nxt):
            xv, lsem, slot = cur
            xv_n, lsem_n, slot_n = nxt

            @pl.when(t + 1 < n_g)
            def _():
                start_load(xv_n, lsem_n, slot_n, t + 1)

            @pl.when(t < n_g)
            def _():
                wait_load(xv, lsem, slot)
                process(xv, slot)

        start_load(xv0, lsem0, 0, 0)

        def pair(p, carry):
            step(2 * p, (xv0, lsem0, 0), (xv1, lsem1, 1))
            step(2 * p + 1, (xv1, lsem1, 1), (xv0, lsem0, 0))
            return carry

        lax.fori_loop(0, (MAXG + 1) // 2, pair, 0)

        @pl.when(wid == NW - 1)
        def _():
            pltpu.sync_copy(x_hbm.at[pl.ds(FULL_GROUPS * GROUP, REM)],
                            xv0.at[pl.ds(0, REM)])
            pltpu.sync_copy(btail_hbm, idx_rem)

            def chunk(c, carry):
                ids_v = idx_rem[pl.ds(c * LANES, LANES)]
                for l in range(LANES):
                    accum_row(xv0, ids_v[l], c * LANES + l)
                return carry

            lax.fori_loop(0, REM // LANES, chunk, 0)

        pltpu.sync_copy(sum_v, sums_o.at[wid])
        pltpu.sync_copy(sq_v, sqs_o.at[wid])
        pltpu.sync_copy(cnt_v, cnts_o.at[wid])

    return k(x, batch, batch_tail)


def _tc_norm(x, batch3, sums, sqs, cnts, w2, b2, ms2):
    def body(x_ref, b_ref, sums_ref, sqs_ref, cnts_ref, w_ref, bi_ref, ms_ref,
             o_ref, hi_scr, lo_scr):
        i = pl.program_id(0)

        @pl.when(i == 0)
        def _():
            sums_c = jnp.sum(sums_ref[...], axis=0)
            sqs_c = jnp.sum(sqs_ref[...], axis=0)
            cnt = jnp.sum(cnts_ref[...], axis=0)[:, 0:1]
            nc = jnp.maximum(cnt, 1.0)
            m = sums_c / nc
            s = ms_ref[...]
            seg_sq = sqs_c - nc * m * m * s * (2.0 - s)
            var = jnp.maximum(seg_sq, 0.0) / nc
            a = w_ref[...] * lax.rsqrt(var + EPS)
            bt = bi_ref[...] - a * s * m
            ab = jnp.concatenate([a, bt], axis=1)
            hi = ab.astype(jnp.bfloat16)
            lo = (ab - hi.astype(jnp.float32)).astype(jnp.bfloat16)
            hi_scr[pl.ds(0, NUM_SEG), :] = hi
            lo_scr[pl.ds(0, NUM_SEG), :] = lo
            pad = jnp.zeros((WPAD, 2 * D), jnp.bfloat16)
            hi_scr[pl.ds(NUM_SEG, WPAD), :] = pad
            lo_scr[pl.ds(NUM_SEG, WPAD), :] = pad

        bm = b_ref[0]
        b0 = b_ref[0, 0, 0]
        blast = b_ref[0, 0, R - 1]
        base16 = pl.multiple_of((b0 >> 4) << 4, 16)
        tdims = (((0,), (0,)), ((), ()))

        @pl.when(blast - base16 < WIN)
        def _():
            oht = (lax.broadcasted_iota(jnp.int32, (WIN, R), 0) + base16 == bm)
            oht = oht.astype(jnp.bfloat16)
            hiw = hi_scr[pl.ds(base16, WIN), :]
            low = lo_scr[pl.ds(base16, WIN), :]
            g = lax.dot_general(oht, hiw, tdims,
                                preferred_element_type=jnp.float32)
            g += lax.dot_general(oht, low, tdims,
                                 preferred_element_type=jnp.float32)
            o_ref[...] = x_ref[...] * g[:, :D] + g[:, D:]

        @pl.when(blast - base16 >= WIN)
        def _():
            oht = (lax.broadcasted_iota(jnp.int32, (NUM_SEG, R), 0) == bm)
            oht = oht.astype(jnp.bfloat16)
            hif = hi_scr[pl.ds(0, NUM_SEG), :]
            lof = lo_scr[pl.ds(0, NUM_SEG), :]
            g = lax.dot_general(oht, hif, tdims,
                                preferred_element_type=jnp.float32)
            g += lax.dot_general(oht, lof, tdims,
                                 preferred_element_type=jnp.float32)
            o_ref[...] = x_ref[...] * g[:, :D] + g[:, D:]

    return pl.pallas_call(
        body,
        grid=(GRID,),
        in_specs=[
            pl.BlockSpec((R, D), lambda i: (i, 0)),
            pl.BlockSpec((1, 1, R), lambda i: (i, 0, 0)),
            pl.BlockSpec((NW, NUM_SEG, D), lambda i: (0, 0, 0)),
            pl.BlockSpec((NW, NUM_SEG, D), lambda i: (0, 0, 0)),
            pl.BlockSpec((NW, NUM_SEG, LANES), lambda i: (0, 0, 0)),
            pl.BlockSpec((1, D), lambda i: (0, 0)),
            pl.BlockSpec((1, D), lambda i: (0, 0)),
            pl.BlockSpec((1, D), lambda i: (0, 0)),
        ],
        out_specs=pl.BlockSpec((R, D), lambda i: (i, 0)),
        out_shape=jax.ShapeDtypeStruct((N, D), jnp.float32),
        scratch_shapes=[
            pltpu.VMEM((NUM_SEG + WPAD, 2 * D), jnp.bfloat16),
            pltpu.VMEM((NUM_SEG + WPAD, 2 * D), jnp.bfloat16),
        ],
        compiler_params=pltpu.CompilerParams(
            dimension_semantics=("arbitrary",)),
    )(x, batch3, sums, sqs, cnts, w2, b2, ms2)


def kernel(x, batch, weight, bias, mean_scale):
    batch_tail = batch[FULL_GROUPS * GROUP:]
    sums, sqs, cnts = _sc_stats(x, batch, batch_tail)
    batch3 = batch.reshape(GRID, 1, R)
    return _tc_norm(x, batch3, sums, sqs, cnts,
                    weight.reshape(1, D), bias.reshape(1, D),
                    mean_scale.reshape(1, D))


# speedup vs baseline: 1.5086x; 1.1246x over previous
import functools

import jax
import jax.numpy as jnp
from jax import lax
from jax.experimental import pallas as pl
from jax.experimental.pallas import tpu as pltpu
from jax.experimental.pallas import tpu_sc as plsc

N = 100000
D = 128
NUM_SEG = 256
EPS = 1e-6
LANES = 16

GROUP = 64
FULL_GROUPS = N // GROUP
REM = N - FULL_GROUPS * GROUP
NW = 32
BASE_G = FULL_GROUPS // NW
EXTRA = FULL_GROUPS - BASE_G * NW
MAXG = BASE_G + 1

R = 4000
GRID = N // R
WIN = 32
WPAD = WIN


def _sc_stats(x, batch, batch_tail):
    mesh = plsc.VectorSubcoreMesh(core_axis_name="c", subcore_axis_name="s")

    @functools.partial(
        pl.kernel,
        mesh=mesh,
        out_type=[
            jax.ShapeDtypeStruct((NW, NUM_SEG, D), jnp.float32),
            jax.ShapeDtypeStruct((NW, NUM_SEG, D), jnp.float32),
            jax.ShapeDtypeStruct((NW, NUM_SEG, LANES), jnp.float32),
        ],
        scratch_types=[
            pltpu.VMEM((GROUP, D), jnp.float32),
            pltpu.VMEM((GROUP, D), jnp.float32),
            pltpu.VMEM((2, GROUP), jnp.int32),
            pltpu.VMEM((REM,), jnp.int32),
            pltpu.VMEM((NUM_SEG, D), jnp.float32),
            pltpu.VMEM((NUM_SEG, D), jnp.float32),
            pltpu.VMEM((NUM_SEG, LANES), jnp.float32),
            pltpu.SemaphoreType.DMA,
            pltpu.SemaphoreType.DMA,
        ],
    )
    def k(x_hbm, b_hbm, btail_hbm, sums_o, sqs_o, cnts_o,
          xv0, xv1, idx2, idx_rem, sum_v, sq_v, cnt_v, lsem0, lsem1):
        cid = lax.axis_index("c")
        sid = lax.axis_index("s")
        wid = cid * 16 + sid

        zero = jnp.zeros((LANES,), jnp.float32)

        def zrow(r, carry):
            cnt_v[r, :] = zero
            for j in range(D // LANES):
                sum_v[r, pl.ds(j * LANES, LANES)] = zero
                sq_v[r, pl.ds(j * LANES, LANES)] = zero
            return carry

        lax.fori_loop(0, NUM_SEG, zrow, 0)

        n_g = jnp.where(wid < EXTRA, MAXG, BASE_G)
        g0 = wid * BASE_G + jnp.minimum(wid, EXTRA)

        def start_load(xv, lsem, slot, t):
            off = (g0 + t) * GROUP
            pltpu.async_copy(x_hbm.at[pl.ds(off, GROUP)], xv, lsem)
            pltpu.async_copy(b_hbm.at[pl.ds(off, GROUP)], idx2.at[slot], lsem)

        def wait_load(xv, lsem, slot):
            pltpu.make_async_copy(x_hbm.at[pl.ds(0, GROUP)], xv, lsem).wait()
            pltpu.make_async_copy(b_hbm.at[pl.ds(0, GROUP)], idx2.at[slot],
                                  lsem).wait()

        def accum_row(xv, seg, r):
            plsc.addupdate(cnt_v.at[seg], jnp.ones((LANES,), jnp.float32))
            for j in range(D // LANES):
                v = xv[r, pl.ds(j * LANES, LANES)]
                plsc.addupdate(sum_v.at[seg, pl.ds(j * LANES, LANES)], v)
                plsc.addupdate(sq_v.at[seg, pl.ds(j * LANES, LANES)], v * v)

        def process(xv, slot):
            ids_head = idx2[slot, pl.ds(0, LANES)]
            ids_tail = idx2[slot, pl.ds(GROUP - LANES, LANES)]
            i0 = ids_head[0]
            i_last = ids_tail[LANES - 1]

            @pl.when(i0 == i_last)
            def _():
                def row(r, accs):
                    out = []
                    for j in range(D // LANES):
                        v = xv[r, pl.ds(j * LANES, LANES)]
                        out.append(accs[2 * j] + v)
                        out.append(accs[2 * j + 1] + v * v)
                    return tuple(out)

                init = tuple(jnp.zeros((LANES,), jnp.float32)
                             for _ in range(2 * (D // LANES)))
                accs = lax.fori_loop(0, GROUP, row, init)
                for j in range(D // LANES):
                    plsc.addupdate(sum_v.at[i0, pl.ds(j * LANES, LANES)],
                                   accs[2 * j])
                    plsc.addupdate(sq_v.at[i0, pl.ds(j * LANES, LANES)],
                                   accs[2 * j + 1])
                plsc.addupdate(cnt_v.at[i0],
                               jnp.full((LANES,), float(GROUP), jnp.float32))

            @pl.when(i0 != i_last)
            def _():
                def chunk(c, carry):
                    ids_v = idx2[slot, pl.ds(c * LANES, LANES)]
                    for l in range(LANES):
                        accum_row(xv, ids_v[l], c * LANES + l)
                    return carry

                lax.fori_loop(0, GROUP // LANES, chunk, 0)

        def step(t, cur, nxt):
            xv, lsem, slot = cur
            xv_n, lsem_n, slot_n = nxt

            @pl.when(t + 1 < n_g)
            def _():
                start_load(xv_n, lsem_n, slot_n, t + 1)

            @pl.when(t < n_g)
            def _():
                wait_load(xv, lsem, slot)
                process(xv, slot)

        start_load(xv0, lsem0, 0, 0)

        def pair(p, carry):
            step(2 * p, (xv0, lsem0, 0), (xv1, lsem1, 1))
            step(2 * p + 1, (xv1, lsem1, 1), (xv0, lsem0, 0))
            return carry

        lax.fori_loop(0, (MAXG + 1) // 2, pair, 0)

        @pl.when(wid == NW - 1)
        def _():
            pltpu.sync_copy(x_hbm.at[pl.ds(FULL_GROUPS * GROUP, REM)],
                            xv0.at[pl.ds(0, REM)])
            pltpu.sync_copy(btail_hbm, idx_rem)

            def chunk(c, carry):
                ids_v = idx_rem[pl.ds(c * LANES, LANES)]
                for l in range(LANES):
                    accum_row(xv0, ids_v[l], c * LANES + l)
                return carry

            lax.fori_loop(0, REM // LANES, chunk, 0)

        pltpu.sync_copy(sum_v, sums_o.at[wid])
        pltpu.sync_copy(sq_v, sqs_o.at[wid])
        pltpu.sync_copy(cnt_v, cnts_o.at[wid])

    return k(x, batch, batch_tail)


def _tc_norm(x, batch3, sums, sqs, cnts, w2, b2, ms2):
    def body(x_ref, b_ref, sums_ref, sqs_ref, cnts_ref, w_ref, bi_ref, ms_ref,
             o_ref, hi_scr, lo_scr):
        i = pl.program_id(0)

        @pl.when(i == 0)
        def _():
            sums_c = jnp.sum(sums_ref[...], axis=0)
            sqs_c = jnp.sum(sqs_ref[...], axis=0)
            cnt = jnp.sum(cnts_ref[...], axis=0)[:, 0:1]
            nc = jnp.maximum(cnt, 1.0)
            m = sums_c / nc
            s = ms_ref[...]
            seg_sq = sqs_c - nc * m * m * s * (2.0 - s)
            var = jnp.maximum(seg_sq, 0.0) / nc
            a = w_ref[...] * lax.rsqrt(var + EPS)
            bt = bi_ref[...] - a * s * m
            ab = jnp.concatenate([a, bt], axis=1)
            hi = ab.astype(jnp.bfloat16)
            lo = (ab - hi.astype(jnp.float32)).astype(jnp.bfloat16)
            hi_scr[pl.ds(0, NUM_SEG), :] = hi
            lo_scr[pl.ds(0, NUM_SEG), :] = lo
            pad = jnp.zeros((WPAD, 2 * D), jnp.bfloat16)
            hi_scr[pl.ds(NUM_SEG, WPAD), :] = pad
            lo_scr[pl.ds(NUM_SEG, WPAD), :] = pad

        bm = b_ref[0]
        b0 = b_ref[0, 0, 0]
        blast = b_ref[0, 0, R - 1]
        base16 = pl.multiple_of((b0 >> 4) << 4, 16)
        tdims = (((0,), (0,)), ((), ()))

        @pl.when(blast - base16 < WIN)
        def _():
            oht = (lax.broadcasted_iota(jnp.int32, (WIN, R), 0) + base16 == bm)
            oht = oht.astype(jnp.bfloat16)
            hiw = hi_scr[pl.ds(base16, WIN), :]
            low = lo_scr[pl.ds(base16, WIN), :]
            g = lax.dot_general(oht, hiw, tdims,
                                preferred_element_type=jnp.float32)
            g += lax.dot_general(oht, low, tdims,
                                 preferred_element_type=jnp.float32)
            o_ref[...] = x_ref[...] * g[:, :D] + g[:, D:]

        @pl.when(blast - base16 >= WIN)
        def _():
            oht = (lax.broadcasted_iota(jnp.int32, (NUM_SEG, R), 0) == bm)
            oht = oht.astype(jnp.bfloat16)
            hif = hi_scr[pl.ds(0, NUM_SEG), :]
            lof = lo_scr[pl.ds(0, NUM_SEG), :]
            g = lax.dot_general(oht, hif, tdims,
                                preferred_element_type=jnp.float32)
            g += lax.dot_general(oht, lof, tdims,
                                 preferred_element_type=jnp.float32)
            o_ref[...] = x_ref[...] * g[:, :D] + g[:, D:]

    return pl.pallas_call(
        body,
        grid=(GRID,),
        in_specs=[
            pl.BlockSpec((R, D), lambda i: (i, 0)),
            pl.BlockSpec((1, 1, R), lambda i: (i, 0, 0)),
            pl.BlockSpec((NW, NUM_SEG, D), lambda i: (0, 0, 0)),
            pl.BlockSpec((NW, NUM_SEG, D), lambda i: (0, 0, 0)),
            pl.BlockSpec((NW, NUM_SEG, LANES), lambda i: (0, 0, 0)),
            pl.BlockSpec((1, D), lambda i: (0, 0)),
            pl.BlockSpec((1, D), lambda i: (0, 0)),
            pl.BlockSpec((1, D), lambda i: (0, 0)),
        ],
        out_specs=pl.BlockSpec((R, D), lambda i: (i, 0)),
        out_shape=jax.ShapeDtypeStruct((N, D), jnp.float32),
        scratch_shapes=[
            pltpu.VMEM((NUM_SEG + WPAD, 2 * D), jnp.bfloat16),
            pltpu.VMEM((NUM_SEG + WPAD, 2 * D), jnp.bfloat16),
        ],
        compiler_params=pltpu.CompilerParams(
            dimension_semantics=("arbitrary",)),
    )(x, batch3, sums, sqs, cnts, w2, b2, ms2)


def kernel(x, batch, weight, bias, mean_scale):
    batch_tail = batch[FULL_GROUPS * GROUP:]
    sums, sqs, cnts = _sc_stats(x, batch, batch_tail)
    batch3 = batch.reshape(GRID, 1, R)
    return _tc_norm(x, batch3, sums, sqs, cnts,
                    weight.reshape(1, D), bias.reshape(1, D),
                    mean_scale.reshape(1, D))


# speedup vs baseline: 1.6210x; 1.0745x over previous
import functools

import jax
import jax.numpy as jnp
from jax import lax
from jax.experimental import pallas as pl
from jax.experimental.pallas import tpu as pltpu
from jax.experimental.pallas import tpu_sc as plsc

N = 100000
D = 128
NUM_SEG = 256
EPS = 1e-6
LANES = 16

GROUP = 64
FULL_GROUPS = N // GROUP
REM = N - FULL_GROUPS * GROUP
NW = 32
BASE_G = FULL_GROUPS // NW
EXTRA = FULL_GROUPS - BASE_G * NW
MAXG = BASE_G + 1

R = 10000
GRID = N // R
WIN = 32
WPAD = WIN


def _sc_stats(x, batch, batch_tail):
    mesh = plsc.VectorSubcoreMesh(core_axis_name="c", subcore_axis_name="s")

    @functools.partial(
        pl.kernel,
        mesh=mesh,
        out_type=[
            jax.ShapeDtypeStruct((NW, NUM_SEG, D), jnp.float32),
            jax.ShapeDtypeStruct((NW, NUM_SEG, D), jnp.float32),
            jax.ShapeDtypeStruct((NW, NUM_SEG, LANES), jnp.float32),
        ],
        scratch_types=[
            pltpu.VMEM((GROUP, D), jnp.float32),
            pltpu.VMEM((GROUP, D), jnp.float32),
            pltpu.VMEM((2, GROUP), jnp.int32),
            pltpu.VMEM((REM,), jnp.int32),
            pltpu.VMEM((NUM_SEG, D), jnp.float32),
            pltpu.VMEM((NUM_SEG, D), jnp.float32),
            pltpu.VMEM((NUM_SEG, LANES), jnp.float32),
            pltpu.SemaphoreType.DMA,
            pltpu.SemaphoreType.DMA,
        ],
    )
    def k(x_hbm, b_hbm, btail_hbm, sums_o, sqs_o, cnts_o,
          xv0, xv1, idx2, idx_rem, sum_v, sq_v, cnt_v, lsem0, lsem1):
        cid = lax.axis_index("c")
        sid = lax.axis_index("s")
        wid = cid * 16 + sid

        zero = jnp.zeros((LANES,), jnp.float32)

        def zrow(r, carry):
            cnt_v[r, :] = zero
            for j in range(D // LANES):
                sum_v[r, pl.ds(j * LANES, LANES)] = zero
                sq_v[r, pl.ds(j * LANES, LANES)] = zero
            return carry

        lax.fori_loop(0, NUM_SEG, zrow, 0)

        n_g = jnp.where(wid < EXTRA, MAXG, BASE_G)
        g0 = wid * BASE_G + jnp.minimum(wid, EXTRA)

        def start_load(xv, lsem, slot, t):
            off = (g0 + t) * GROUP
            pltpu.async_copy(x_hbm.at[pl.ds(off, GROUP)], xv, lsem)
            pltpu.async_copy(b_hbm.at[pl.ds(off, GROUP)], idx2.at[slot], lsem)

        def wait_load(xv, lsem, slot):
            pltpu.make_async_copy(x_hbm.at[pl.ds(0, GROUP)], xv, lsem).wait()
            pltpu.make_async_copy(b_hbm.at[pl.ds(0, GROUP)], idx2.at[slot],
                                  lsem).wait()

        def accum_row(xv, seg, r):
            plsc.addupdate(cnt_v.at[seg], jnp.ones((LANES,), jnp.float32))
            for j in range(D // LANES):
                v = xv[r, pl.ds(j * LANES, LANES)]
                plsc.addupdate(sum_v.at[seg, pl.ds(j * LANES, LANES)], v)
                plsc.addupdate(sq_v.at[seg, pl.ds(j * LANES, LANES)], v * v)

        def process(xv, slot):
            ids_head = idx2[slot, pl.ds(0, LANES)]
            ids_tail = idx2[slot, pl.ds(GROUP - LANES, LANES)]
            i0 = ids_head[0]
            i_last = ids_tail[LANES - 1]

            @pl.when(i0 == i_last)
            def _():
                def row(r, accs):
                    out = []
                    for j in range(D // LANES):
                        v = xv[r, pl.ds(j * LANES, LANES)]
                        out.append(accs[2 * j] + v)
                        out.append(accs[2 * j + 1] + v * v)
                    return tuple(out)

                init = tuple(jnp.zeros((LANES,), jnp.float32)
                             for _ in range(2 * (D // LANES)))
                accs = lax.fori_loop(0, GROUP, row, init)
                for j in range(D // LANES):
                    plsc.addupdate(sum_v.at[i0, pl.ds(j * LANES, LANES)],
                                   accs[2 * j])
                    plsc.addupdate(sq_v.at[i0, pl.ds(j * LANES, LANES)],
                                   accs[2 * j + 1])
                plsc.addupdate(cnt_v.at[i0],
                               jnp.full((LANES,), float(GROUP), jnp.float32))

            @pl.when(i0 != i_last)
            def _():
                def chunk(c, carry):
                    ids_v = idx2[slot, pl.ds(c * LANES, LANES)]
                    for l in range(LANES):
                        accum_row(xv, ids_v[l], c * LANES + l)
                    return carry

                lax.fori_loop(0, GROUP // LANES, chunk, 0)

        def step(t, cur, nxt):
            xv, lsem, slot = cur
            xv_n, lsem_n, slot_n = nxt

            @pl.when(t + 1 < n_g)
            def _():
                start_load(xv_n, lsem_n, slot_n, t + 1)

            @pl.when(t < n_g)
            def _():
                wait_load(xv, lsem, slot)
                process(xv, slot)

        start_load(xv0, lsem0, 0, 0)

        def pair(p, carry):
            step(2 * p, (xv0, lsem0, 0), (xv1, lsem1, 1))
            step(2 * p + 1, (xv1, lsem1, 1), (xv0, lsem0, 0))
            return carry

        lax.fori_loop(0, (MAXG + 1) // 2, pair, 0)

        @pl.when(wid == NW - 1)
        def _():
            pltpu.sync_copy(x_hbm.at[pl.ds(FULL_GROUPS * GROUP, REM)],
                            xv0.at[pl.ds(0, REM)])
            pltpu.sync_copy(btail_hbm, idx_rem)

            def chunk(c, carry):
                ids_v = idx_rem[pl.ds(c * LANES, LANES)]
                for l in range(LANES):
                    accum_row(xv0, ids_v[l], c * LANES + l)
                return carry

            lax.fori_loop(0, REM // LANES, chunk, 0)

        pltpu.sync_copy(sum_v, sums_o.at[wid])
        pltpu.sync_copy(sq_v, sqs_o.at[wid])
        pltpu.sync_copy(cnt_v, cnts_o.at[wid])

    return k(x, batch, batch_tail)


def _tc_norm(x, batch3, sums, sqs, cnts, w2, b2, ms2):
    def body(x_ref, b_ref, sums_ref, sqs_ref, cnts_ref, w_ref, bi_ref, ms_ref,
             o_ref, hi_scr, lo_scr):
        i = pl.program_id(0)

        @pl.when(i == 0)
        def _():
            sums_c = jnp.sum(sums_ref[...], axis=0)
            sqs_c = jnp.sum(sqs_ref[...], axis=0)
            cnt = jnp.sum(cnts_ref[...], axis=0)[:, 0:1]
            nc = jnp.maximum(cnt, 1.0)
            m = sums_c / nc
            s = ms_ref[...]
            seg_sq = sqs_c - nc * m * m * s * (2.0 - s)
            var = jnp.maximum(seg_sq, 0.0) / nc
            a = w_ref[...] * lax.rsqrt(var + EPS)
            bt = bi_ref[...] - a * s * m
            ab = jnp.concatenate([a, bt], axis=1)
            hi = ab.astype(jnp.bfloat16)
            lo = (ab - hi.astype(jnp.float32)).astype(jnp.bfloat16)
            hi_scr[pl.ds(0, NUM_SEG), :] = hi
            lo_scr[pl.ds(0, NUM_SEG), :] = lo
            pad = jnp.zeros((WPAD, 2 * D), jnp.bfloat16)
            hi_scr[pl.ds(NUM_SEG, WPAD), :] = pad
            lo_scr[pl.ds(NUM_SEG, WPAD), :] = pad

        bm = b_ref[0]
        b0 = b_ref[0, 0, 0]
        blast = b_ref[0, 0, R - 1]
        base16 = pl.multiple_of((b0 >> 4) << 4, 16)
        tdims = (((0,), (0,)), ((), ()))

        @pl.when(blast - base16 < WIN)
        def _():
            oht = (lax.broadcasted_iota(jnp.int32, (WIN, R), 0) + base16 == bm)
            oht = oht.astype(jnp.bfloat16)
            hiw = hi_scr[pl.ds(base16, WIN), :]
            low = lo_scr[pl.ds(base16, WIN), :]
            g = lax.dot_general(oht, hiw, tdims,
                                preferred_element_type=jnp.float32)
            g += lax.dot_general(oht, low, tdims,
                                 preferred_element_type=jnp.float32)
            o_ref[...] = x_ref[...] * g[:, :D] + g[:, D:]

        @pl.when(blast - base16 >= WIN)
        def _():
            oht = (lax.broadcasted_iota(jnp.int32, (NUM_SEG, R), 0) == bm)
            oht = oht.astype(jnp.bfloat16)
            hif = hi_scr[pl.ds(0, NUM_SEG), :]
            lof = lo_scr[pl.ds(0, NUM_SEG), :]
            g = lax.dot_general(oht, hif, tdims,
                                preferred_element_type=jnp.float32)
            g += lax.dot_general(oht, lof, tdims,
                                 preferred_element_type=jnp.float32)
            o_ref[...] = x_ref[...] * g[:, :D] + g[:, D:]

    return pl.pallas_call(
        body,
        grid=(GRID,),
        in_specs=[
            pl.BlockSpec((R, D), lambda i: (i, 0)),
            pl.BlockSpec((1, 1, R), lambda i: (i, 0, 0)),
            pl.BlockSpec((NW, NUM_SEG, D), lambda i: (0, 0, 0)),
            pl.BlockSpec((NW, NUM_SEG, D), lambda i: (0, 0, 0)),
            pl.BlockSpec((NW, NUM_SEG, LANES), lambda i: (0, 0, 0)),
            pl.BlockSpec((1, D), lambda i: (0, 0)),
            pl.BlockSpec((1, D), lambda i: (0, 0)),
            pl.BlockSpec((1, D), lambda i: (0, 0)),
        ],
        out_specs=pl.BlockSpec((R, D), lambda i: (i, 0)),
        out_shape=jax.ShapeDtypeStruct((N, D), jnp.float32),
        scratch_shapes=[
            pltpu.VMEM((NUM_SEG + WPAD, 2 * D), jnp.bfloat16),
            pltpu.VMEM((NUM_SEG + WPAD, 2 * D), jnp.bfloat16),
        ],
        compiler_params=pltpu.CompilerParams(
            dimension_semantics=("arbitrary",)),
    )(x, batch3, sums, sqs, cnts, w2, b2, ms2)


def kernel(x, batch, weight, bias, mean_scale):
    batch_tail = batch[FULL_GROUPS * GROUP:]
    sums, sqs, cnts = _sc_stats(x, batch, batch_tail)
    batch3 = batch.reshape(GRID, 1, R)
    return _tc_norm(x, batch3, sums, sqs, cnts,
                    weight.reshape(1, D), bias.reshape(1, D),
                    mean_scale.reshape(1, D))
